# baseline (speedup 1.0000x reference)
import jax
import jax.numpy as jnp
from jax import lax
from jax.experimental import pallas as pl
from jax.experimental.pallas import tpu as pltpu

MASKS_SIZE = 64
GRID_W = 6
C_CHUNK = 24


def _mul_body(m_ref, x_ref, o_ref):
    b = pl.program_id(0)
    col = lax.broadcasted_iota(jnp.int32, (384,), 0) // MASKS_SIZE
    for r in range(6):
        cell = col + r * GRID_W
        keep = jnp.ones((384,), dtype=jnp.bool_)
        for k in range(4):
            keep = jnp.logical_and(keep, cell != m_ref[b, k])
        m = keep.astype(jnp.float32)[None, None, :]
        band = slice(r * MASKS_SIZE, (r + 1) * MASKS_SIZE)
        o_ref[0, :, band, :] = x_ref[0, :, band, :] * m


def kernel(x, m_indices):
    b, c, h, w = x.shape
    grid = (b, c // C_CHUNK)
    return pl.pallas_call(
        _mul_body,
        grid=grid,
        in_specs=[
            pl.BlockSpec(memory_space=pltpu.SMEM),
            pl.BlockSpec((1, C_CHUNK, h, w), lambda i, j: (i, j, 0, 0)),
        ],
        out_specs=pl.BlockSpec((1, C_CHUNK, h, w), lambda i, j: (i, j, 0, 0)),
        out_shape=jax.ShapeDtypeStruct(x.shape, x.dtype),
    )(m_indices, x)
